# Initial kernel scaffold; baseline (speedup 1.0000x reference)
import jax
import jax.numpy as jnp
from jax.experimental import pallas as pl


def kernel(h, edge_index, k_cross_channel, K_channel_wise):
    raise NotImplementedError("write your pallas kernel here")


# speedup vs baseline: 23.5554x; 23.5554x over previous
import functools

import jax
import jax.numpy as jnp
from jax import lax
from jax.experimental import pallas as pl
from jax.experimental.pallas import tpu as pltpu
from jax.experimental.pallas import tpu_sc as plsc

N = 10000
D = 128
E = 320000

NW = 32
EPB = 128
CPT = 79
EPAD = NW * CPT * EPB
NROWS = 10112
RPT = NROWS // 16
NDEG = 10240
DPT = NDEG // 16

_mesh = plsc.VectorSubcoreMesh(core_axis_name="c", subcore_axis_name="s")


@functools.partial(
    pl.kernel,
    mesh=_mesh,
    out_type=jax.ShapeDtypeStruct((2 * NDEG,), jnp.float32),
    scratch_types=[
        pltpu.VMEM((CPT, EPB), jnp.int32),
        pltpu.VMEM((EPB,), jnp.float32),
        pltpu.VMEM((DPT,), jnp.float32),
        pltpu.VMEM_SHARED((NDEG,), jnp.float32),
    ],
)
def _sc_deg(dst_hbm, out_hbm, idx_v, ones_v, zb_v, deg_sh):
    c = lax.axis_index("c")
    s = lax.axis_index("s")
    wid = c * 16 + s
    for i in range(EPB // 16):
        ones_v[pl.ds(i * 16, 16)] = jnp.ones((16,), jnp.float32)
    for i in range(DPT // 16):
        zb_v[pl.ds(i * 16, 16)] = jnp.zeros((16,), jnp.float32)
    pltpu.sync_copy(zb_v, deg_sh.at[pl.ds(s * DPT, DPT)])
    plsc.subcore_barrier()
    pltpu.sync_copy(dst_hbm.at[wid], idx_v)

    def body(j, carry):
        pltpu.sync_copy(ones_v, deg_sh.at[idx_v.at[j]], add=True)
        return carry

    lax.fori_loop(0, CPT, body, 0)
    plsc.subcore_barrier()
    pltpu.sync_copy(deg_sh.at[pl.ds(s * DPT, DPT)],
                    out_hbm.at[pl.ds(c * NDEG + s * DPT, DPT)])


@functools.partial(
    pl.kernel,
    mesh=_mesh,
    out_type=jax.ShapeDtypeStruct((2, NROWS, D), jnp.float32),
    scratch_types=[
        pltpu.VMEM((CPT, EPB), jnp.int32),
        pltpu.VMEM((CPT, EPB), jnp.int32),
        pltpu.VMEM((EPB, D), jnp.float32),
        pltpu.VMEM_SHARED((NROWS, D), jnp.float32),
        pltpu.SemaphoreType.DMA,
    ],
)
def _sc_spmm(g_hbm, src_hbm, dst_hbm, out_hbm, si_v, di_v, rows_v,
             acc_sh, sem):
    c = lax.axis_index("c")
    s = lax.axis_index("s")
    wid = c * 16 + s

    def zrow(j, carry):
        for i in range(D // 16):
            rows_v[j, pl.ds(i * 16, 16)] = jnp.zeros((16,), jnp.float32)
        return carry

    lax.fori_loop(0, EPB, zrow, 0)
    for b in range(4):
        pltpu.sync_copy(rows_v, acc_sh.at[pl.ds(s * RPT + b * EPB, EPB)])
    pltpu.sync_copy(rows_v.at[pl.ds(0, RPT - 4 * EPB)],
                    acc_sh.at[pl.ds(s * RPT + 4 * EPB, RPT - 4 * EPB)])
    pltpu.sync_copy(src_hbm.at[wid], si_v)
    pltpu.sync_copy(dst_hbm.at[wid], di_v)
    plsc.subcore_barrier()

    def body(j, carry):
        pltpu.async_copy(g_hbm.at[di_v.at[j]], rows_v, sem).wait()
        pltpu.sync_copy(rows_v, acc_sh.at[si_v.at[j]], add=True)
        return carry

    lax.fori_loop(0, CPT, body, 0)
    plsc.subcore_barrier()
    pltpu.sync_copy(acc_sh.at[pl.ds(s * RPT, RPT)],
                    out_hbm.at[c, pl.ds(s * RPT, RPT)])


_RB = 2528


def _tc_g_body(h_ref, d0_ref, d1_ref, g_ref, dis_ref):
    dis = lax.rsqrt(d0_ref[...] + d1_ref[...] + 1.0)
    g_ref[...] = h_ref[...] * dis
    dis_ref[...] = dis


def _tc_g(h_pad, d0, d1):
    grid = (NROWS // _RB,)
    return pl.pallas_call(
        _tc_g_body,
        grid=grid,
        in_specs=[
            pl.BlockSpec((_RB, D), lambda i: (i, 0)),
            pl.BlockSpec((_RB, 1), lambda i: (i, 0)),
            pl.BlockSpec((_RB, 1), lambda i: (i, 0)),
        ],
        out_specs=[
            pl.BlockSpec((_RB, D), lambda i: (i, 0)),
            pl.BlockSpec((_RB, 1), lambda i: (i, 0)),
        ],
        out_shape=[
            jax.ShapeDtypeStruct((NROWS, D), jnp.float32),
            jax.ShapeDtypeStruct((NROWS, 1), jnp.float32),
        ],
    )(h_pad, d0, d1)


def _tc_final_body(h_ref, g_ref, acc_ref, dis_ref, c1_ref, c2_ref, o_ref):
    accsum = acc_ref[0] + acc_ref[1]
    a = dis_ref[...] * (accsum + g_ref[...])
    o_ref[...] = h_ref[...] * c1_ref[...] + a * c2_ref[...]


def _tc_final(h_pad, g_pad, acc, dis_col, c1, c2):
    grid = (NROWS // _RB,)
    return pl.pallas_call(
        _tc_final_body,
        grid=grid,
        in_specs=[
            pl.BlockSpec((_RB, D), lambda i: (i, 0)),
            pl.BlockSpec((_RB, D), lambda i: (i, 0)),
            pl.BlockSpec((2, _RB, D), lambda i: (0, i, 0)),
            pl.BlockSpec((_RB, 1), lambda i: (i, 0)),
            pl.BlockSpec((1, D), lambda i: (0, 0)),
            pl.BlockSpec((1, D), lambda i: (0, 0)),
        ],
        out_specs=pl.BlockSpec((_RB, D), lambda i: (i, 0)),
        out_shape=jax.ShapeDtypeStruct((NROWS, D), jnp.float32),
    )(h_pad, g_pad, acc, dis_col, c1, c2)


def kernel(h, edge_index, k_cross_channel, K_channel_wise):
    src = edge_index[0].astype(jnp.int32)
    dst = edge_index[1].astype(jnp.int32)
    pad = jnp.full((EPAD - E,), N, jnp.int32)
    srcp = jnp.concatenate([src, pad]).reshape(NW, CPT, EPB)
    dstp = jnp.concatenate([dst, pad]).reshape(NW, CPT, EPB)
    h_pad = jnp.pad(h, ((0, NROWS - N), (0, 0)))

    deg_flat = _sc_deg(dstp)
    d0 = deg_flat[:NROWS, None]
    d1 = deg_flat[NDEG:NDEG + NROWS, None]
    g_pad, dis_col = _tc_g(h_pad, d0, d1)
    acc = _sc_spmm(g_pad, srcp, dstp)

    k = k_cross_channel[0]
    c1 = (2.0 - k - K_channel_wise) * 0.5
    c2 = (k + K_channel_wise) * 0.5
    out_pad = _tc_final(h_pad, g_pad, acc, dis_col, c1, c2)
    return out_pad[:N]
